# baseline (speedup 1.0000x reference)
import functools

import jax
import jax.numpy as jnp
from jax import lax
from jax.experimental import pallas as pl
from jax.experimental.pallas import tpu as pltpu

N = 10000
E = 160000
D = 256


def _mm_body(relu_in, relu_mid, two, x_ref, w1_ref, b1_ref, w2_ref, b2_ref,
             o_ref):
    x = x_ref[...]
    if relu_in:
        x = jnp.maximum(x, 0.0)
    t = jnp.dot(x, w1_ref[...], preferred_element_type=jnp.float32) + b1_ref[...]
    if relu_mid:
        t = jnp.maximum(t, 0.0)
    if two:
        t = jnp.dot(t, w2_ref[...], preferred_element_type=jnp.float32) + b2_ref[...]
    o_ref[...] = t


def _mm(x, w1, b1, w2=None, b2=None, relu_in=False, relu_mid=False,
        row_block=1000):
    n, k = x.shape
    m1 = w1.shape[1]
    two = w2 is not None
    m = w2.shape[1] if two else m1
    if not two:
        w2 = jnp.zeros((8, 128), jnp.float32)
        b2 = jnp.zeros((128,), jnp.float32)
    grid = n // row_block
    body = functools.partial(_mm_body, relu_in, relu_mid, two)
    return pl.pallas_call(
        body,
        grid=(grid,),
        in_specs=[
            pl.BlockSpec((row_block, k), lambda i: (i, 0)),
            pl.BlockSpec(w1.shape, lambda i: (0, 0)),
            pl.BlockSpec(b1.shape, lambda i: (0,)),
            pl.BlockSpec(w2.shape, lambda i: (0, 0)),
            pl.BlockSpec(b2.shape, lambda i: (0,)),
        ],
        out_specs=pl.BlockSpec((row_block, m), lambda i: (i, 0)),
        out_shape=jax.ShapeDtypeStruct((n, m), jnp.float32),
    )(x, w1, b1, w2, b2)


def _att_pass1(src, dst, r, a1, a2):
    t = a1[src] + r + a2[dst]
    e = jnp.where(t > 0, t, 0.01 * t)
    xs = jnp.exp(e)
    den = jax.ops.segment_sum(xs, dst, num_segments=N)
    return xs, den


def _agg_pair(src_ss, dst_ss, al_ss, P_ss, f_ss,
              src_os, dst_os, al_os, P_os, f_os):
    agg = (jax.ops.segment_sum(al_ss[:, None] * P_ss[src_ss], dst_ss, num_segments=N)
           + jax.ops.segment_sum(al_os[:, None] * P_os[src_os], dst_os, num_segments=N))
    S_ss = jax.ops.segment_sum(al_ss[:, None] * f_ss, dst_ss, num_segments=N)
    S_os = jax.ops.segment_sum(al_os[:, None] * f_os, dst_os, num_segments=N)
    return agg, S_ss, S_os


def _gather_scatter(src, dst, H):
    return jax.ops.segment_sum(H[src], dst, num_segments=N)


def _decode(so_src, so_dst, Z, X):
    return Z[so_src] * X[so_dst]


def kernel(s_feat, o_feat, ss_edges, ss_feat, os_edges, os_feat, fwd_edges,
           bwd_edges, so_edges, W_s, b_s, W_os, b_os, W_ss, b_ss, W_attn,
           b_attn, W_in, b_in, W_self, b_self, W_out, b_out, W_o, b_o):
    f32 = jnp.float32
    src_ss, dst_ss = ss_edges[0], ss_edges[1]
    src_os, dst_os = os_edges[0], os_edges[1]

    Wa1 = [W_attn[l][:D] for l in range(2)]
    Wa2 = [W_attn[l][D:] for l in range(2)]
    Wss_top = [W_ss[l][:D] for l in range(2)]
    Wss_bot = [W_ss[l][D:] for l in range(2)]
    Wos_top = [W_os[l][:D] for l in range(2)]
    Wos_bot = [W_os[l][D:] for l in range(2)]

    def pad128(cols):
        z = jnp.zeros((cols[0].shape[0], 128), f32)
        for i, c in enumerate(cols):
            z = z.at[:, i].set(c)
        return z

    Ws_big, bs_big = [], []
    for l in range(2):
        wa1 = (Wss_top[l] @ Wa1[l])[:, 0]
        wa2 = (W_s[l] @ Wa2[l])[:, 0]
        Ws_big.append(jnp.concatenate([Wss_top[l], pad128([wa1, wa2])], axis=1))
        sc = jnp.zeros((128,), f32).at[0].set(b_ss[l] @ Wa1[l][:, 0]) \
            .at[1].set(b_s[l] @ Wa2[l][:, 0] + b_attn[l][0])
        bs_big.append(jnp.concatenate([b_ss[l], sc]))
    Wo_big, bo_big = [], []
    for l in range(2):
        wa1 = (Wos_top[l] @ Wa1[l])[:, 0]
        Wo_big.append(jnp.concatenate(
            [Wos_top[l], W_in[l], W_self[l], W_out[l], pad128([wa1])], axis=1))
        sc = jnp.zeros((128,), f32).at[0].set(b_os[l] @ Wa1[l][:, 0])
        bo_big.append(jnp.concatenate([b_os[l], b_in[l], b_self[l], b_out[l], sc]))
    rW = jnp.zeros((32, 128), f32)
    for l in range(2):
        rW = rW.at[:10, l].set((Wss_bot[l] @ Wa1[l])[:, 0])
        rW = rW.at[16:18, 2 + l].set((Wos_bot[l] @ Wa1[l])[:, 0])
    Wz = []
    for l in range(2):
        wb1 = jnp.zeros((16, D), f32).at[:10].set(Wss_bot[l])
        wb2 = jnp.zeros((16, D), f32).at[:2].set(Wos_bot[l])
        Wz.append(jnp.concatenate([jnp.eye(D, dtype=f32), wb1, wb2], axis=0))
    zeroD = jnp.zeros((D,), f32)

    ss_f16 = jnp.pad(ss_feat, ((0, 0), (0, 6)))
    os_f16 = jnp.pad(os_feat, ((0, 0), (0, 14)))

    r_all = _mm(jnp.concatenate([ss_f16, os_f16], axis=1), rW,
                jnp.zeros((128,), f32), row_block=2000)
    r_ss = [r_all[:, 0], r_all[:, 1]]
    r_os = [r_all[:, 2], r_all[:, 3]]

    sf, of = s_feat, o_feat
    agg_prev = S_ss_prev = S_os_prev = None
    for l in range(2):
        if l == 0:
            sp = _mm(sf, Ws_big[l], bs_big[l])
            op = _mm(of, Wo_big[l], bo_big[l])
        else:
            xz = jnp.concatenate([agg_prev, S_ss_prev, S_os_prev], axis=1)
            sp = _mm(xz, Wz[l - 1], zeroD, Ws_big[l], bs_big[l], relu_mid=True)
            op = _mm(of, Wo_big[l], bo_big[l], relu_in=True)
        P_ss, a1_ss, a2 = sp[:, :D], sp[:, D], sp[:, D + 1]
        P_os, a1_os = op[:, :D], op[:, 4 * D]
        h_in, h_self, h_out = op[:, D:2 * D], op[:, 2 * D:3 * D], op[:, 3 * D:4 * D]

        xs_ss, den_ss = _att_pass1(src_ss, dst_ss, r_ss[l], a1_ss, a2)
        xs_os, den_os = _att_pass1(src_os, dst_os, r_os[l], a1_os, a2)
        al_ss = xs_ss / den_ss[dst_ss]
        al_os = xs_os / den_os[dst_os]
        agg, S_ss, S_os = _agg_pair(src_ss, dst_ss, al_ss, P_ss, ss_f16,
                                    src_os, dst_os, al_os, P_os, os_f16)

        h_in_agg = _gather_scatter(fwd_edges[0], fwd_edges[1], h_in)
        h_out_agg = _gather_scatter(bwd_edges[0], bwd_edges[1], h_out)
        x = _mm(jnp.concatenate([h_in_agg, h_self, h_out_agg], axis=1),
                jnp.concatenate([W_o[l][:D], W_o[l][D:2 * D], W_o[l][2 * D:]],
                                axis=0), b_o[l], relu_in=True)
        agg_prev, S_ss_prev, S_os_prev = agg, S_ss, S_os
        of = x

    Z = _mm(jnp.concatenate([agg_prev, S_ss_prev, S_os_prev], axis=1),
            Wz[1], zeroD)
    return _decode(so_edges[0], so_edges[1], Z, of)


# speedup vs baseline: 2.6931x; 2.6931x over previous
import functools

import jax
import jax.numpy as jnp
from jax import lax
from jax.experimental import pallas as pl
from jax.experimental.pallas import tpu as pltpu
from jax.experimental.pallas import tpu_sc as plsc

N = 10000
E = 160000
D = 256

_MESH = plsc.VectorSubcoreMesh(core_axis_name="c", subcore_axis_name="s",
                               num_cores=2, num_subcores=16)
_NSUB = 16
_CH = 128
_EP = 163840
_EPS = _EP // _NSUB
_EPW = _EP // 32
_NP = 10240
_RPS = _NP // _NSUB


def _mm_body(relu_in, relu_mid, two, x_ref, w1_ref, b1_ref, w2_ref, b2_ref,
             o_ref):
    x = x_ref[...]
    if relu_in:
        x = jnp.maximum(x, 0.0)
    t = jnp.dot(x, w1_ref[...], preferred_element_type=jnp.float32) + b1_ref[...]
    if relu_mid:
        t = jnp.maximum(t, 0.0)
    if two:
        t = jnp.dot(t, w2_ref[...], preferred_element_type=jnp.float32) + b2_ref[...]
    o_ref[...] = t


def _mm(x, w1, b1, w2=None, b2=None, relu_in=False, relu_mid=False,
        row_block=1000):
    n, k = x.shape
    m1 = w1.shape[1]
    two = w2 is not None
    m = w2.shape[1] if two else m1
    if not two:
        w2 = jnp.zeros((8, 128), jnp.float32)
        b2 = jnp.zeros((128,), jnp.float32)
    grid = n // row_block
    body = functools.partial(_mm_body, relu_in, relu_mid, two)
    return pl.pallas_call(
        body,
        grid=(grid,),
        in_specs=[
            pl.BlockSpec((row_block, k), lambda i: (i, 0)),
            pl.BlockSpec(w1.shape, lambda i: (0, 0)),
            pl.BlockSpec(b1.shape, lambda i: (0,)),
            pl.BlockSpec(w2.shape, lambda i: (0, 0)),
            pl.BlockSpec(b2.shape, lambda i: (0,)),
        ],
        out_specs=pl.BlockSpec((row_block, m), lambda i: (i, 0)),
        out_shape=jax.ShapeDtypeStruct((n, m), jnp.float32),
    )(x, w1, b1, w2, b2)


def _sc_att_pass1(src_ss, dst_ss, r_ss, a1_ss, a2, src_os, dst_os, r_os,
                  a1_os):
    @functools.partial(
        pl.kernel,
        out_type=[jax.ShapeDtypeStruct((_EP,), jnp.float32),
                  jax.ShapeDtypeStruct((_EP,), jnp.float32),
                  jax.ShapeDtypeStruct((2 * _NP,), jnp.float32),
                  jax.ShapeDtypeStruct((2 * _NP,), jnp.float32)],
        name="sc_att1",
        mesh=_MESH,
        scratch_types=[
            pltpu.VMEM((_CH,), jnp.int32),
            pltpu.VMEM((_CH,), jnp.int32),
            pltpu.VMEM((_CH,), jnp.float32),
            pltpu.VMEM((_CH,), jnp.float32),
            pltpu.VMEM((_CH,), jnp.float32),
            pltpu.VMEM((_CH,), jnp.float32),
            pltpu.VMEM_SHARED((_NP,), jnp.float32),
            pltpu.VMEM_SHARED((_NP,), jnp.float32),
            pltpu.SemaphoreType.DMA,
        ],
    )
    def k(ss_s_hbm, ss_d_hbm, rss_hbm, a1ss_hbm, a2_hbm, os_s_hbm, os_d_hbm,
          ros_hbm, a1os_hbm, xss_hbm, xos_hbm, denss_hbm, denos_hbm,
          si_v, di_v, r_v, a1g_v, a2g_v, xs_v, denss_sh, denos_sh, sem):
        c = lax.axis_index("c")
        s = lax.axis_index("s")
        base = (s * 2 + c) * _EPW

        @pl.loop(0, _CH // 16)
        def _(j):
            xs_v[pl.ds(j * 16, 16)] = jnp.zeros((16,), jnp.float32)
        for acc in (denss_sh, denos_sh):
            for j in range(_RPS // _CH):
                pltpu.sync_copy(xs_v, acc.at[pl.ds(s * _RPS + j * _CH, _CH)])
        plsc.subcore_barrier()

        def one_type(s_hbm, d_hbm, rr_hbm, a1_hbm, xs_hbm, den_sh):
            @pl.loop(0, _EPW // _CH)
            def _(t):
                off = base + t * _CH
                pltpu.sync_copy(s_hbm.at[pl.ds(off, _CH)], si_v)
                pltpu.sync_copy(d_hbm.at[pl.ds(off, _CH)], di_v)
                pltpu.sync_copy(rr_hbm.at[pl.ds(off, _CH)], r_v)
                pltpu.async_copy(a1_hbm.at[si_v], a1g_v, sem).wait()
                pltpu.async_copy(a2_hbm.at[di_v], a2g_v, sem).wait()

                @pl.loop(0, _CH // 16)
                def _(j):
                    sl = pl.ds(j * 16, 16)
                    tv = a1g_v[sl] + r_v[sl] + a2g_v[sl]
                    tv = jnp.where(tv > 0, tv, 0.01 * tv)
                    xs_v[sl] = jnp.exp(tv)
                pltpu.sync_copy(xs_v, xs_hbm.at[pl.ds(off, _CH)])
                pltpu.sync_copy(xs_v, den_sh.at[di_v], add=True)

        one_type(ss_s_hbm, ss_d_hbm, rss_hbm, a1ss_hbm, xss_hbm, denss_sh)
        one_type(os_s_hbm, os_d_hbm, ros_hbm, a1os_hbm, xos_hbm, denos_sh)
        plsc.subcore_barrier()
        pltpu.sync_copy(denss_sh.at[pl.ds(s * _RPS, _RPS)],
                        denss_hbm.at[pl.ds(c * _NP + s * _RPS, _RPS)])
        pltpu.sync_copy(denos_sh.at[pl.ds(s * _RPS, _RPS)],
                        denos_hbm.at[pl.ds(c * _NP + s * _RPS, _RPS)])

    return k(src_ss, dst_ss, r_ss, a1_ss, a2, src_os, dst_os, r_os, a1_os)


def _tc_densum(den2):
    x = den2.reshape(2, _NP // 128, 128)

    def body(x_ref, o_ref):
        o_ref[...] = x_ref[0] + x_ref[1]

    return pl.pallas_call(
        body,
        grid=(_NP // 1024,),
        in_specs=[pl.BlockSpec((2, 8, 128), lambda i: (0, i, 0))],
        out_specs=pl.BlockSpec((8, 128), lambda i: (i, 0)),
        out_shape=jax.ShapeDtypeStruct((_NP // 128, 128), jnp.float32),
    )(x).reshape(_NP)


def _sc_scale_rows(rows_ref, al_ref, width16):
    @pl.loop(0, _CH // 16)
    def _(j):
        av = al_ref[pl.ds(j * 16, 16)]
        for m in range(16):
            a = av[m]
            for k8 in range(width16):
                sl = pl.ds(k8 * 16, 16)
                rows_ref[j * 16 + m, sl] = rows_ref[j * 16 + m, sl] * a


def _sc_zero_rows(rows_v, ncols16):
    @pl.loop(0, _CH)
    def _(i):
        for k in range(ncols16):
            rows_v[i, pl.ds(k * 16, 16)] = jnp.zeros((16,), jnp.float32)


def _sc_zero_shared(rows_v, acc_sh, s):
    nfull, rem = _RPS // _CH, _RPS % _CH
    for j in range(nfull):
        pltpu.sync_copy(rows_v, acc_sh.at[pl.ds(s * _RPS + j * _CH, _CH)])
    if rem:
        pltpu.sync_copy(rows_v.at[pl.ds(0, rem)],
                        acc_sh.at[pl.ds(s * _RPS + nfull * _CH, rem)])


def _sc_dump_shared(acc_sh, out_hbm, c, s):
    pltpu.sync_copy(acc_sh.at[pl.ds(s * _RPS, _RPS)],
                    out_hbm.at[c, pl.ds(s * _RPS, _RPS)])


def _sc_att_agg(src_ss, dst_ss, xs_ss, den2_ss, P_ss, q_ss,
                src_os, dst_os, xs_os, den2_os, P_os, q_os):
    Pss2 = jnp.pad(P_ss, ((0, 8), (0, 0))).reshape(2 * N + 16, 128)
    Pos2 = jnp.pad(P_os, ((0, 8), (0, 0))).reshape(2 * N + 16, 128)
    den_ss = _tc_densum(den2_ss)
    den_os = _tc_densum(den2_os)

    @functools.partial(
        pl.kernel,
        out_type=jax.ShapeDtypeStruct((2, _NP, 128), jnp.float32),
        name="sc_attagg",
        mesh=_MESH,
        scratch_types=[
            pltpu.VMEM((_CH,), jnp.int32),
            pltpu.VMEM((_CH,), jnp.int32),
            pltpu.VMEM((_CH,), jnp.int32),
            pltpu.VMEM((_CH,), jnp.float32),
            pltpu.VMEM((_CH,), jnp.float32),
            pltpu.VMEM((_CH, 128), jnp.float32),
            pltpu.VMEM((_CH, 128), jnp.float32),
            pltpu.VMEM_SHARED((_NP, 128), jnp.float32),
            pltpu.SemaphoreType.DMA,
        ],
    )
    def k(sss_hbm, ssd_hbm, xss_hbm, denss_hbm, pss_hbm, qss_hbm,
          oss_hbm, osd_hbm, xos_hbm, denos_hbm, pos_hbm, qos_hbm,
          agg_hbm, si_v, di_v, ai_v, xs_v, al_v, rows_v, rows2_v, agg_sh,
          sem):
        c = lax.axis_index("c")
        s = lax.axis_index("s")
        base = s * _EPS

        def one_type(s_hbm_, d_hbm_, x_hbm_, p_hbm_, q_hbm_, den_hbm_):
            @pl.loop(0, _EPS // _CH)
            def _(t):
                off = base + t * _CH
                pltpu.sync_copy(s_hbm_.at[pl.ds(off, _CH)], si_v)
                pltpu.sync_copy(d_hbm_.at[pl.ds(off, _CH)], di_v)
                pltpu.sync_copy(x_hbm_.at[pl.ds(off, _CH)], xs_v)
                pltpu.async_copy(den_hbm_.at[di_v], al_v, sem).wait()

                @pl.loop(0, _CH // 16)
                def _(j):
                    sl = pl.ds(j * 16, 16)
                    ai_v[sl] = si_v[sl] * 2 + c
                    al_v[sl] = xs_v[sl] / al_v[sl]
                pltpu.async_copy(p_hbm_.at[ai_v], rows_v, sem).wait()
                pltpu.sync_copy(q_hbm_.at[pl.ds(c * _EP + off, _CH)], rows2_v)
                _sc_scale_rows(rows_v, al_v, 8)
                _sc_scale_rows(rows2_v, al_v, 8)
                pltpu.sync_copy(rows_v, agg_sh.at[di_v], add=True)
                pltpu.sync_copy(rows2_v, agg_sh.at[di_v], add=True)

        _sc_zero_rows(rows_v, 8)
        _sc_zero_shared(rows_v, agg_sh, s)
        plsc.subcore_barrier()
        one_type(sss_hbm, ssd_hbm, xss_hbm, pss_hbm, qss_hbm, denss_hbm)
        one_type(oss_hbm, osd_hbm, xos_hbm, pos_hbm, qos_hbm, denos_hbm)
        plsc.subcore_barrier()
        _sc_dump_shared(agg_sh, agg_hbm, c, s)

    agg2 = k(src_ss, dst_ss, xs_ss, den_ss, Pss2, q_ss,
             src_os, dst_os, xs_os, den_os, Pos2, q_os)
    return jnp.transpose(agg2[:, :N], (1, 0, 2)).reshape(N, 256)


def _sc_dual_agg(Hin, Hout, fsrc, fdst, bsrc, bdst):
    Hin2 = jnp.pad(Hin, ((0, 8), (0, 0))).reshape(2 * N + 16, 128)
    Hout2 = jnp.pad(Hout, ((0, 8), (0, 0))).reshape(2 * N + 16, 128)

    @functools.partial(
        pl.kernel,
        out_type=[jax.ShapeDtypeStruct((2, _NP, 128), jnp.float32),
                  jax.ShapeDtypeStruct((2, _NP, 128), jnp.float32)],
        name="sc_dualagg",
        mesh=_MESH,
        scratch_types=[
            pltpu.VMEM((_CH,), jnp.int32),
            pltpu.VMEM((_CH,), jnp.int32),
            pltpu.VMEM((_CH,), jnp.int32),
            pltpu.VMEM((_CH, 128), jnp.float32),
            pltpu.VMEM_SHARED((_NP, 128), jnp.float32),
            pltpu.SemaphoreType.DMA,
        ],
    )
    def k(hin_hbm, hout_hbm, fs_hbm, fd_hbm, bs_hbm, bd_hbm,
          oin_hbm, oout_hbm, si_v, di_v, ai_v, rows_v, acc_sh, sem):
        c = lax.axis_index("c")
        s = lax.axis_index("s")
        base = s * _EPS

        def one_pass(tab_hbm, src_hbm, dst_hbm):
            @pl.loop(0, _EPS // _CH)
            def _(t):
                off = base + t * _CH
                pltpu.sync_copy(src_hbm.at[pl.ds(off, _CH)], si_v)
                pltpu.sync_copy(dst_hbm.at[pl.ds(off, _CH)], di_v)

                @pl.loop(0, _CH // 16)
                def _(j):
                    sl = pl.ds(j * 16, 16)
                    ai_v[sl] = si_v[sl] * 2 + c
                pltpu.async_copy(tab_hbm.at[ai_v], rows_v, sem).wait()
                pltpu.sync_copy(rows_v, acc_sh.at[di_v], add=True)

        _sc_zero_rows(rows_v, 8)
        _sc_zero_shared(rows_v, acc_sh, s)
        plsc.subcore_barrier()
        one_pass(hin_hbm, fs_hbm, fd_hbm)
        plsc.subcore_barrier()
        _sc_dump_shared(acc_sh, oin_hbm, c, s)
        plsc.subcore_barrier()
        _sc_zero_rows(rows_v, 8)
        _sc_zero_shared(rows_v, acc_sh, s)
        plsc.subcore_barrier()
        one_pass(hout_hbm, bs_hbm, bd_hbm)
        plsc.subcore_barrier()
        _sc_dump_shared(acc_sh, oout_hbm, c, s)

    oin, oout = k(Hin2, Hout2, fsrc, fdst, bsrc, bdst)
    to_n256 = lambda o: jnp.transpose(o[:, :N], (1, 0, 2)).reshape(N, 256)
    return to_n256(oin), to_n256(oout)


def _decode(so_src, so_dst, Z, X):
    @functools.partial(
        pl.kernel,
        out_type=jax.ShapeDtypeStruct((_EP, 256), jnp.float32),
        name="sc_decode",
        mesh=_MESH,
        scratch_types=[
            pltpu.VMEM((_CH,), jnp.int32),
            pltpu.VMEM((_CH,), jnp.int32),
            pltpu.VMEM((_CH, 256), jnp.float32),
            pltpu.VMEM((_CH, 256), jnp.float32),
            pltpu.SemaphoreType.DMA,
        ],
    )
    def k(z_hbm, x_hbm, si_hbm, di_hbm, out_hbm, si_v, di_v, za_v, xb_v, sem):
        c = lax.axis_index("c")
        s = lax.axis_index("s")
        base = (s * 2 + c) * _EPW

        @pl.loop(0, _EPW // _CH)
        def _(t):
            off = base + t * _CH
            pltpu.sync_copy(si_hbm.at[pl.ds(off, _CH)], si_v)
            pltpu.sync_copy(di_hbm.at[pl.ds(off, _CH)], di_v)
            pltpu.async_copy(z_hbm.at[si_v], za_v, sem).wait()
            pltpu.async_copy(x_hbm.at[di_v], xb_v, sem).wait()

            @pl.loop(0, _CH)
            def _(i):
                for k8 in range(16):
                    sl = pl.ds(k8 * 16, 16)
                    za_v[i, sl] = za_v[i, sl] * xb_v[i, sl]
            pltpu.sync_copy(za_v, out_hbm.at[pl.ds(off, _CH)])

    Zp = jnp.pad(Z, ((0, 8), (0, 0)))
    Xp = jnp.pad(X, ((0, 8), (0, 0)))
    return k(Zp, Xp, so_src, so_dst)[:E]


def kernel(s_feat, o_feat, ss_edges, ss_feat, os_edges, os_feat, fwd_edges,
           bwd_edges, so_edges, W_s, b_s, W_os, b_os, W_ss, b_ss, W_attn,
           b_attn, W_in, b_in, W_self, b_self, W_out, b_out, W_o, b_o):
    f32 = jnp.float32
    i32 = jnp.int32

    def epad(e2):
        return jnp.pad(e2.astype(i32), ((0, 0), (0, _EP - E)),
                       constant_values=N)

    ss_edges = epad(ss_edges)
    os_edges = epad(os_edges)
    fwd_edges = epad(fwd_edges)
    bwd_edges = epad(bwd_edges)
    so_edges = epad(so_edges)
    src_ss, dst_ss = ss_edges[0], ss_edges[1]
    src_os, dst_os = os_edges[0], os_edges[1]

    Wa1 = [W_attn[l][:D] for l in range(2)]
    Wa2 = [W_attn[l][D:] for l in range(2)]
    Wss_top = [W_ss[l][:D] for l in range(2)]
    Wss_bot = [W_ss[l][D:] for l in range(2)]
    Wos_top = [W_os[l][:D] for l in range(2)]
    Wos_bot = [W_os[l][D:] for l in range(2)]

    def pad128(cols):
        z = jnp.zeros((cols[0].shape[0], 128), f32)
        for i, c in enumerate(cols):
            z = z.at[:, i].set(c)
        return z

    Ws_big, bs_big = [], []
    for l in range(2):
        wa1 = (Wss_top[l] @ Wa1[l])[:, 0]
        wa2 = (W_s[l] @ Wa2[l])[:, 0]
        Ws_big.append(jnp.concatenate([Wss_top[l], pad128([wa1, wa2])], axis=1))
        sc = jnp.zeros((128,), f32).at[0].set(b_ss[l] @ Wa1[l][:, 0]) \
            .at[1].set(b_s[l] @ Wa2[l][:, 0] + b_attn[l][0])
        bs_big.append(jnp.concatenate([b_ss[l], sc]))
    Wo_big, bo_big = [], []
    for l in range(2):
        wa1 = (Wos_top[l] @ Wa1[l])[:, 0]
        Wo_big.append(jnp.concatenate(
            [Wos_top[l], W_in[l], W_self[l], W_out[l], pad128([wa1])], axis=1))
        sc = jnp.zeros((128,), f32).at[0].set(b_os[l] @ Wa1[l][:, 0])
        bo_big.append(jnp.concatenate([b_os[l], b_in[l], b_self[l], b_out[l], sc]))
    rW = jnp.zeros((32, 128), f32)
    for l in range(2):
        rW = rW.at[:10, l].set((Wss_bot[l] @ Wa1[l])[:, 0])
        rW = rW.at[16:18, 2 + l].set((Wos_bot[l] @ Wa1[l])[:, 0])
    Wq_ss = [jnp.zeros((16, D), f32).at[:10].set(Wss_bot[l]) for l in range(2)]
    Wq_os = [jnp.zeros((16, D), f32).at[:2].set(Wos_bot[l]) for l in range(2)]
    zeroD = jnp.zeros((D,), f32)

    ss_f16 = jnp.pad(ss_feat, ((0, 0), (0, 6)))
    os_f16 = jnp.pad(os_feat, ((0, 0), (0, 14)))

    r_all = _mm(jnp.concatenate([ss_f16, os_f16], axis=1), rW,
                jnp.zeros((128,), f32), row_block=2000)
    rpad = lambda v: jnp.pad(v, (0, _EP - E))
    r_ss = [rpad(r_all[:, 0]), rpad(r_all[:, 1])]
    r_os = [rpad(r_all[:, 2]), rpad(r_all[:, 3])]
    ss_f16p = jnp.pad(ss_f16, ((0, _EP - E), (0, 0)))
    os_f16p = jnp.pad(os_f16, ((0, _EP - E), (0, 0)))

    sf, of = s_feat, o_feat
    agg_prev = None
    for l in range(2):
        if l == 0:
            sp = _mm(sf, Ws_big[l], bs_big[l])
            op = _mm(of, Wo_big[l], bo_big[l])
        else:
            sp = _mm(agg_prev, Ws_big[l], bs_big[l], relu_in=True)
            op = _mm(of, Wo_big[l], bo_big[l], relu_in=True)
        P_ss, a1_ss, a2 = sp[:, :D], sp[:, D], sp[:, D + 1]
        P_os, a1_os = op[:, :D], op[:, 4 * D]
        h_in, h_self, h_out = op[:, D:2 * D], op[:, 2 * D:3 * D], op[:, 3 * D:4 * D]

        def qcat(qe):
            qp = jnp.pad(qe, ((0, _EP - E), (0, 0)))
            return jnp.concatenate([qp[:, :128], qp[:, 128:]], axis=0)
        q_ss = qcat(_mm(ss_f16, Wq_ss[l], zeroD, row_block=2000))
        q_os = qcat(_mm(os_f16, Wq_os[l], zeroD, row_block=2000))

        npad = lambda v: jnp.pad(v, (0, _NP - N))
        xs_ss, xs_os, den2_ss, den2_os = _sc_att_pass1(
            src_ss, dst_ss, r_ss[l], npad(a1_ss), npad(a2), src_os, dst_os,
            r_os[l], npad(a1_os))
        agg = _sc_att_agg(src_ss, dst_ss, xs_ss, den2_ss, P_ss, q_ss,
                          src_os, dst_os, xs_os, den2_os, P_os, q_os)

        h_in_agg, h_out_agg = _sc_dual_agg(h_in, h_out, fwd_edges[0],
                                           fwd_edges[1], bwd_edges[0],
                                           bwd_edges[1])
        x = _mm(jnp.concatenate([h_in_agg, h_self, h_out_agg], axis=1),
                jnp.concatenate([W_o[l][:D], W_o[l][D:2 * D], W_o[l][2 * D:]],
                                axis=0), b_o[l], relu_in=True)
        agg_prev = agg
        of = x

    return _decode(so_edges[0], so_edges[1], agg_prev, of)


# speedup vs baseline: 2.9719x; 1.1035x over previous
import functools

import jax
import jax.numpy as jnp
from jax import lax
from jax.experimental import pallas as pl
from jax.experimental.pallas import tpu as pltpu
from jax.experimental.pallas import tpu_sc as plsc

N = 10000
E = 160000
D = 256

_MESH = plsc.VectorSubcoreMesh(core_axis_name="c", subcore_axis_name="s",
                               num_cores=2, num_subcores=16)
_NSUB = 16
_CH = 128
_EP = 163840
_EPS = _EP // _NSUB
_EPW = _EP // 32
_NP = 10240
_RPS = _NP // _NSUB


def _mm_body(relu_in, relu_mid, two, x_ref, w1_ref, b1_ref, w2_ref, b2_ref,
             o_ref):
    x = x_ref[...]
    if relu_in:
        x = jnp.maximum(x, 0.0)
    t = jnp.dot(x, w1_ref[...], preferred_element_type=jnp.float32) + b1_ref[...]
    if relu_mid:
        t = jnp.maximum(t, 0.0)
    if two:
        t = jnp.dot(t, w2_ref[...], preferred_element_type=jnp.float32) + b2_ref[...]
    o_ref[...] = t


def _mm(x, w1, b1, w2=None, b2=None, relu_in=False, relu_mid=False,
        row_block=1000):
    n, k = x.shape
    m1 = w1.shape[1]
    two = w2 is not None
    m = w2.shape[1] if two else m1
    if not two:
        w2 = jnp.zeros((8, 128), jnp.float32)
        b2 = jnp.zeros((128,), jnp.float32)
    grid = n // row_block
    body = functools.partial(_mm_body, relu_in, relu_mid, two)
    return pl.pallas_call(
        body,
        grid=(grid,),
        in_specs=[
            pl.BlockSpec((row_block, k), lambda i: (i, 0)),
            pl.BlockSpec(w1.shape, lambda i: (0, 0)),
            pl.BlockSpec(b1.shape, lambda i: (0,)),
            pl.BlockSpec(w2.shape, lambda i: (0, 0)),
            pl.BlockSpec(b2.shape, lambda i: (0,)),
        ],
        out_specs=pl.BlockSpec((row_block, m), lambda i: (i, 0)),
        out_shape=jax.ShapeDtypeStruct((n, m), jnp.float32),
    )(x, w1, b1, w2, b2)


def _sc_att_pass1(src_ss, dst_ss, r_ss, a1_ss, a2, src_os, dst_os, r_os,
                  a1_os):
    @functools.partial(
        pl.kernel,
        out_type=[jax.ShapeDtypeStruct((_EP,), jnp.float32),
                  jax.ShapeDtypeStruct((_EP,), jnp.float32),
                  jax.ShapeDtypeStruct((2 * _NP,), jnp.float32),
                  jax.ShapeDtypeStruct((2 * _NP,), jnp.float32)],
        name="sc_att1",
        mesh=_MESH,
        scratch_types=[
            pltpu.VMEM((_CH,), jnp.int32),
            pltpu.VMEM((_CH,), jnp.int32),
            pltpu.VMEM((_CH,), jnp.float32),
            pltpu.VMEM((_CH,), jnp.float32),
            pltpu.VMEM((_CH,), jnp.float32),
            pltpu.VMEM((_CH,), jnp.float32),
            pltpu.VMEM_SHARED((_NP,), jnp.float32),
            pltpu.VMEM_SHARED((_NP,), jnp.float32),
            pltpu.SemaphoreType.DMA,
        ],
    )
    def k(ss_s_hbm, ss_d_hbm, rss_hbm, a1ss_hbm, a2_hbm, os_s_hbm, os_d_hbm,
          ros_hbm, a1os_hbm, xss_hbm, xos_hbm, denss_hbm, denos_hbm,
          si_v, di_v, r_v, a1g_v, a2g_v, xs_v, denss_sh, denos_sh, sem):
        c = lax.axis_index("c")
        s = lax.axis_index("s")
        base = (s * 2 + c) * _EPW

        @pl.loop(0, _CH // 16)
        def _(j):
            xs_v[pl.ds(j * 16, 16)] = jnp.zeros((16,), jnp.float32)
        for acc in (denss_sh, denos_sh):
            for j in range(_RPS // _CH):
                pltpu.sync_copy(xs_v, acc.at[pl.ds(s * _RPS + j * _CH, _CH)])
        plsc.subcore_barrier()

        def one_type(s_hbm, d_hbm, rr_hbm, a1_hbm, xs_hbm, den_sh):
            @pl.loop(0, _EPW // _CH)
            def _(t):
                off = base + t * _CH
                pltpu.sync_copy(s_hbm.at[pl.ds(off, _CH)], si_v)
                pltpu.sync_copy(d_hbm.at[pl.ds(off, _CH)], di_v)
                pltpu.sync_copy(rr_hbm.at[pl.ds(off, _CH)], r_v)
                cp1 = pltpu.async_copy(a1_hbm.at[si_v], a1g_v, sem)
                cp2 = pltpu.async_copy(a2_hbm.at[di_v], a2g_v, sem)
                cp1.wait()
                cp2.wait()

                @pl.loop(0, _CH // 16)
                def _(j):
                    sl = pl.ds(j * 16, 16)
                    tv = a1g_v[sl] + r_v[sl] + a2g_v[sl]
                    tv = jnp.where(tv > 0, tv, 0.01 * tv)
                    xs_v[sl] = jnp.exp(tv)
                pltpu.sync_copy(xs_v, xs_hbm.at[pl.ds(off, _CH)])
                pltpu.sync_copy(xs_v, den_sh.at[di_v], add=True)

        one_type(ss_s_hbm, ss_d_hbm, rss_hbm, a1ss_hbm, xss_hbm, denss_sh)
        one_type(os_s_hbm, os_d_hbm, ros_hbm, a1os_hbm, xos_hbm, denos_sh)
        plsc.subcore_barrier()
        pltpu.sync_copy(denss_sh.at[pl.ds(s * _RPS, _RPS)],
                        denss_hbm.at[pl.ds(c * _NP + s * _RPS, _RPS)])
        pltpu.sync_copy(denos_sh.at[pl.ds(s * _RPS, _RPS)],
                        denos_hbm.at[pl.ds(c * _NP + s * _RPS, _RPS)])

    return k(src_ss, dst_ss, r_ss, a1_ss, a2, src_os, dst_os, r_os, a1_os)


def _tc_densum(den2):
    x = den2.reshape(2, _NP // 128, 128)

    def body(x_ref, o_ref):
        o_ref[...] = x_ref[0] + x_ref[1]

    return pl.pallas_call(
        body,
        grid=(_NP // 1024,),
        in_specs=[pl.BlockSpec((2, 8, 128), lambda i: (0, i, 0))],
        out_specs=pl.BlockSpec((8, 128), lambda i: (i, 0)),
        out_shape=jax.ShapeDtypeStruct((_NP // 128, 128), jnp.float32),
    )(x).reshape(_NP)


def _sc_scale_rows(rows_ref, al_ref, width16):
    @pl.loop(0, _CH // 16)
    def _(j):
        av = al_ref[pl.ds(j * 16, 16)]
        for m in range(16):
            a = av[m]
            for k8 in range(width16):
                sl = pl.ds(k8 * 16, 16)
                rows_ref[j * 16 + m, sl] = rows_ref[j * 16 + m, sl] * a


def _sc_zero_rows(rows_v, ncols16):
    @pl.loop(0, _CH)
    def _(i):
        for k in range(ncols16):
            rows_v[i, pl.ds(k * 16, 16)] = jnp.zeros((16,), jnp.float32)


def _sc_zero_shared(rows_v, acc_sh, s):
    nfull, rem = _RPS // _CH, _RPS % _CH
    for j in range(nfull):
        pltpu.sync_copy(rows_v, acc_sh.at[pl.ds(s * _RPS + j * _CH, _CH)])
    if rem:
        pltpu.sync_copy(rows_v.at[pl.ds(0, rem)],
                        acc_sh.at[pl.ds(s * _RPS + nfull * _CH, rem)])


def _sc_dump_shared(acc_sh, out_hbm, c, s):
    pltpu.sync_copy(acc_sh.at[pl.ds(s * _RPS, _RPS)],
                    out_hbm.at[c, pl.ds(s * _RPS, _RPS)])


def _sc_att_agg(src_ss, dst_ss, xs_ss, den2_ss, P_ss, q_ss,
                src_os, dst_os, xs_os, den2_os, P_os, q_os):
    Pss2 = jnp.pad(P_ss, ((0, 8), (0, 0))).reshape(2 * N + 16, 128)
    Pos2 = jnp.pad(P_os, ((0, 8), (0, 0))).reshape(2 * N + 16, 128)
    den_ss = _tc_densum(den2_ss)
    den_os = _tc_densum(den2_os)

    @functools.partial(
        pl.kernel,
        out_type=jax.ShapeDtypeStruct((2, _NP, 128), jnp.float32),
        name="sc_attagg",
        mesh=_MESH,
        scratch_types=[
            pltpu.VMEM((_CH,), jnp.int32),
            pltpu.VMEM((_CH,), jnp.int32),
            pltpu.VMEM((_CH,), jnp.int32),
            pltpu.VMEM((_CH,), jnp.float32),
            pltpu.VMEM((_CH,), jnp.float32),
            pltpu.VMEM((_CH, 128), jnp.float32),
            pltpu.VMEM((_CH, 128), jnp.float32),
            pltpu.VMEM_SHARED((_NP, 128), jnp.float32),
            pltpu.SemaphoreType.DMA,
        ],
    )
    def k(sss_hbm, ssd_hbm, xss_hbm, denss_hbm, pss_hbm, qss_hbm,
          oss_hbm, osd_hbm, xos_hbm, denos_hbm, pos_hbm, qos_hbm,
          agg_hbm, si_v, di_v, ai_v, xs_v, al_v, rows_v, rows2_v, agg_sh,
          sem):
        c = lax.axis_index("c")
        s = lax.axis_index("s")
        base = s * _EPS

        def one_type(s_hbm_, d_hbm_, x_hbm_, p_hbm_, q_hbm_, den_hbm_):
            @pl.loop(0, _EPS // _CH)
            def _(t):
                off = base + t * _CH
                pltpu.sync_copy(s_hbm_.at[pl.ds(off, _CH)], si_v)
                pltpu.sync_copy(d_hbm_.at[pl.ds(off, _CH)], di_v)
                pltpu.sync_copy(x_hbm_.at[pl.ds(off, _CH)], xs_v)
                cpden = pltpu.async_copy(den_hbm_.at[di_v], al_v, sem)

                @pl.loop(0, _CH // 16)
                def _(j):
                    sl = pl.ds(j * 16, 16)
                    ai_v[sl] = si_v[sl] * 2 + c
                cprows = pltpu.async_copy(p_hbm_.at[ai_v], rows_v, sem)
                pltpu.sync_copy(q_hbm_.at[pl.ds(c * _EP + off, _CH)], rows2_v)
                cpden.wait()

                @pl.loop(0, _CH // 16)
                def _(j):
                    sl = pl.ds(j * 16, 16)
                    al_v[sl] = xs_v[sl] / al_v[sl]
                cprows.wait()
                _sc_scale_rows(rows_v, al_v, 8)
                _sc_scale_rows(rows2_v, al_v, 8)
                pltpu.sync_copy(rows_v, agg_sh.at[di_v], add=True)
                pltpu.sync_copy(rows2_v, agg_sh.at[di_v], add=True)

        _sc_zero_rows(rows_v, 8)
        _sc_zero_shared(rows_v, agg_sh, s)
        plsc.subcore_barrier()
        one_type(sss_hbm, ssd_hbm, xss_hbm, pss_hbm, qss_hbm, denss_hbm)
        one_type(oss_hbm, osd_hbm, xos_hbm, pos_hbm, qos_hbm, denos_hbm)
        plsc.subcore_barrier()
        _sc_dump_shared(agg_sh, agg_hbm, c, s)

    agg2 = k(src_ss, dst_ss, xs_ss, den_ss, Pss2, q_ss,
             src_os, dst_os, xs_os, den_os, Pos2, q_os)
    return jnp.transpose(agg2[:, :N], (1, 0, 2)).reshape(N, 256)


def _sc_dual_agg(Hin, Hout, fsrc, fdst, bsrc, bdst):
    Hin2 = jnp.pad(Hin, ((0, 8), (0, 0))).reshape(2 * N + 16, 128)
    Hout2 = jnp.pad(Hout, ((0, 8), (0, 0))).reshape(2 * N + 16, 128)

    @functools.partial(
        pl.kernel,
        out_type=[jax.ShapeDtypeStruct((2, _NP, 128), jnp.float32),
                  jax.ShapeDtypeStruct((2, _NP, 128), jnp.float32)],
        name="sc_dualagg",
        mesh=_MESH,
        scratch_types=[
            pltpu.VMEM((_CH,), jnp.int32),
            pltpu.VMEM((_CH,), jnp.int32),
            pltpu.VMEM((_CH,), jnp.int32),
            pltpu.VMEM((_CH, 128), jnp.float32),
            pltpu.VMEM_SHARED((_NP, 128), jnp.float32),
            pltpu.SemaphoreType.DMA,
        ],
    )
    def k(hin_hbm, hout_hbm, fs_hbm, fd_hbm, bs_hbm, bd_hbm,
          oin_hbm, oout_hbm, si_v, di_v, ai_v, rows_v, acc_sh, sem):
        c = lax.axis_index("c")
        s = lax.axis_index("s")
        base = s * _EPS

        def one_pass(tab_hbm, src_hbm, dst_hbm):
            @pl.loop(0, _EPS // _CH)
            def _(t):
                off = base + t * _CH
                pltpu.sync_copy(src_hbm.at[pl.ds(off, _CH)], si_v)
                pltpu.sync_copy(dst_hbm.at[pl.ds(off, _CH)], di_v)

                @pl.loop(0, _CH // 16)
                def _(j):
                    sl = pl.ds(j * 16, 16)
                    ai_v[sl] = si_v[sl] * 2 + c
                pltpu.async_copy(tab_hbm.at[ai_v], rows_v, sem).wait()
                pltpu.sync_copy(rows_v, acc_sh.at[di_v], add=True)

        _sc_zero_rows(rows_v, 8)
        _sc_zero_shared(rows_v, acc_sh, s)
        plsc.subcore_barrier()
        one_pass(hin_hbm, fs_hbm, fd_hbm)
        plsc.subcore_barrier()
        _sc_dump_shared(acc_sh, oin_hbm, c, s)
        plsc.subcore_barrier()
        _sc_zero_rows(rows_v, 8)
        _sc_zero_shared(rows_v, acc_sh, s)
        plsc.subcore_barrier()
        one_pass(hout_hbm, bs_hbm, bd_hbm)
        plsc.subcore_barrier()
        _sc_dump_shared(acc_sh, oout_hbm, c, s)

    oin, oout = k(Hin2, Hout2, fsrc, fdst, bsrc, bdst)
    to_n256 = lambda o: jnp.transpose(o[:, :N], (1, 0, 2)).reshape(N, 256)
    return to_n256(oin), to_n256(oout)


def _decode(so_src, so_dst, Z, X):
    @functools.partial(
        pl.kernel,
        out_type=jax.ShapeDtypeStruct((_EP, 256), jnp.float32),
        name="sc_decode",
        mesh=_MESH,
        scratch_types=[
            pltpu.VMEM((_CH,), jnp.int32),
            pltpu.VMEM((_CH,), jnp.int32),
            pltpu.VMEM((_CH, 256), jnp.float32),
            pltpu.VMEM((_CH, 256), jnp.float32),
            pltpu.SemaphoreType.DMA,
        ],
    )
    def k(z_hbm, x_hbm, si_hbm, di_hbm, out_hbm, si_v, di_v, za_v, xb_v, sem):
        c = lax.axis_index("c")
        s = lax.axis_index("s")
        base = (s * 2 + c) * _EPW

        @pl.loop(0, _EPW // _CH)
        def _(t):
            off = base + t * _CH
            pltpu.sync_copy(si_hbm.at[pl.ds(off, _CH)], si_v)
            pltpu.sync_copy(di_hbm.at[pl.ds(off, _CH)], di_v)
            cpa = pltpu.async_copy(z_hbm.at[si_v], za_v, sem)
            cpb = pltpu.async_copy(x_hbm.at[di_v], xb_v, sem)
            cpa.wait()
            cpb.wait()

            @pl.loop(0, _CH)
            def _(i):
                for k8 in range(16):
                    sl = pl.ds(k8 * 16, 16)
                    za_v[i, sl] = za_v[i, sl] * xb_v[i, sl]
            pltpu.sync_copy(za_v, out_hbm.at[pl.ds(off, _CH)])

    Zp = jnp.pad(Z, ((0, 8), (0, 0)))
    Xp = jnp.pad(X, ((0, 8), (0, 0)))
    return k(Zp, Xp, so_src, so_dst)[:E]


def kernel(s_feat, o_feat, ss_edges, ss_feat, os_edges, os_feat, fwd_edges,
           bwd_edges, so_edges, W_s, b_s, W_os, b_os, W_ss, b_ss, W_attn,
           b_attn, W_in, b_in, W_self, b_self, W_out, b_out, W_o, b_o):
    f32 = jnp.float32
    i32 = jnp.int32

    def epad(e2):
        return jnp.pad(e2.astype(i32), ((0, 0), (0, _EP - E)),
                       constant_values=N)

    ss_edges = epad(ss_edges)
    os_edges = epad(os_edges)
    fwd_edges = epad(fwd_edges)
    bwd_edges = epad(bwd_edges)
    so_edges = epad(so_edges)
    src_ss, dst_ss = ss_edges[0], ss_edges[1]
    src_os, dst_os = os_edges[0], os_edges[1]

    Wa1 = [W_attn[l][:D] for l in range(2)]
    Wa2 = [W_attn[l][D:] for l in range(2)]
    Wss_top = [W_ss[l][:D] for l in range(2)]
    Wss_bot = [W_ss[l][D:] for l in range(2)]
    Wos_top = [W_os[l][:D] for l in range(2)]
    Wos_bot = [W_os[l][D:] for l in range(2)]

    def pad128(cols):
        z = jnp.zeros((cols[0].shape[0], 128), f32)
        for i, c in enumerate(cols):
            z = z.at[:, i].set(c)
        return z

    Ws_big, bs_big = [], []
    for l in range(2):
        wa1 = (Wss_top[l] @ Wa1[l])[:, 0]
        wa2 = (W_s[l] @ Wa2[l])[:, 0]
        Ws_big.append(jnp.concatenate([Wss_top[l], pad128([wa1, wa2])], axis=1))
        sc = jnp.zeros((128,), f32).at[0].set(b_ss[l] @ Wa1[l][:, 0]) \
            .at[1].set(b_s[l] @ Wa2[l][:, 0] + b_attn[l][0])
        bs_big.append(jnp.concatenate([b_ss[l], sc]))
    Wo_big, bo_big = [], []
    for l in range(2):
        wa1 = (Wos_top[l] @ Wa1[l])[:, 0]
        Wo_big.append(jnp.concatenate(
            [Wos_top[l], W_in[l], W_self[l], W_out[l], pad128([wa1])], axis=1))
        sc = jnp.zeros((128,), f32).at[0].set(b_os[l] @ Wa1[l][:, 0])
        bo_big.append(jnp.concatenate([b_os[l], b_in[l], b_self[l], b_out[l], sc]))
    rW = jnp.zeros((32, 128), f32)
    for l in range(2):
        rW = rW.at[:10, l].set((Wss_bot[l] @ Wa1[l])[:, 0])
        rW = rW.at[16:18, 2 + l].set((Wos_bot[l] @ Wa1[l])[:, 0])
    Wq_ss = [jnp.zeros((16, D), f32).at[:10].set(Wss_bot[l]) for l in range(2)]
    Wq_os = [jnp.zeros((16, D), f32).at[:2].set(Wos_bot[l]) for l in range(2)]
    zeroD = jnp.zeros((D,), f32)

    ss_f16 = jnp.pad(ss_feat, ((0, 0), (0, 6)))
    os_f16 = jnp.pad(os_feat, ((0, 0), (0, 14)))

    r_all = _mm(jnp.concatenate([ss_f16, os_f16], axis=1), rW,
                jnp.zeros((128,), f32), row_block=2000)
    rpad = lambda v: jnp.pad(v, (0, _EP - E))
    r_ss = [rpad(r_all[:, 0]), rpad(r_all[:, 1])]
    r_os = [rpad(r_all[:, 2]), rpad(r_all[:, 3])]
    ss_f16p = jnp.pad(ss_f16, ((0, _EP - E), (0, 0)))
    os_f16p = jnp.pad(os_f16, ((0, _EP - E), (0, 0)))

    sf, of = s_feat, o_feat
    agg_prev = None
    for l in range(2):
        if l == 0:
            sp = _mm(sf, Ws_big[l], bs_big[l])
            op = _mm(of, Wo_big[l], bo_big[l])
        else:
            sp = _mm(agg_prev, Ws_big[l], bs_big[l], relu_in=True)
            op = _mm(of, Wo_big[l], bo_big[l], relu_in=True)
        P_ss, a1_ss, a2 = sp[:, :D], sp[:, D], sp[:, D + 1]
        P_os, a1_os = op[:, :D], op[:, 4 * D]
        h_in, h_self, h_out = op[:, D:2 * D], op[:, 2 * D:3 * D], op[:, 3 * D:4 * D]

        def qcat(qe):
            qp = jnp.pad(qe, ((0, _EP - E), (0, 0)))
            return jnp.concatenate([qp[:, :128], qp[:, 128:]], axis=0)
        q_ss = qcat(_mm(ss_f16, Wq_ss[l], zeroD, row_block=2000))
        q_os = qcat(_mm(os_f16, Wq_os[l], zeroD, row_block=2000))

        npad = lambda v: jnp.pad(v, (0, _NP - N))
        xs_ss, xs_os, den2_ss, den2_os = _sc_att_pass1(
            src_ss, dst_ss, r_ss[l], npad(a1_ss), npad(a2), src_os, dst_os,
            r_os[l], npad(a1_os))
        agg = _sc_att_agg(src_ss, dst_ss, xs_ss, den2_ss, P_ss, q_ss,
                          src_os, dst_os, xs_os, den2_os, P_os, q_os)

        h_in_agg, h_out_agg = _sc_dual_agg(h_in, h_out, fwd_edges[0],
                                           fwd_edges[1], bwd_edges[0],
                                           bwd_edges[1])
        x = _mm(jnp.concatenate([h_in_agg, h_self, h_out_agg], axis=1),
                jnp.concatenate([W_o[l][:D], W_o[l][D:2 * D], W_o[l][2 * D:]],
                                axis=0), b_o[l], relu_in=True)
        agg_prev = agg
        of = x

    return _decode(so_edges[0], so_edges[1], agg_prev, of)
